# baseline (speedup 1.0000x reference)
import functools

import jax
import jax.numpy as jnp
from jax import lax
from jax.experimental import pallas as pl
from jax.experimental.pallas import tpu as pltpu
from jax.experimental.pallas import tpu_sc as plsc

_GAMMA = 12.0
_B = 16384
_D = 128
_LANES = 16
_CHUNK = 128


def _sc_geometry():
    try:
        info = plsc.get_sparse_core_info()
        return info.num_cores, info.num_subcores
    except Exception:
        return 2, 16


def _kge_body(nc, bpw, ent_hbm, rel_hbm, hidx_hbm, ridx_hbm, tidx_hbm,
              out_hbm, hidx_v, ridx_v, tidx_v, hrows, rrows, trows,
              out_v, sem):
    wid = lax.axis_index("s") * nc + lax.axis_index("c")
    base = wid * bpw
    pltpu.sync_copy(hidx_hbm.at[pl.ds(base, bpw)], hidx_v)
    pltpu.sync_copy(ridx_hbm.at[pl.ds(base, bpw)], ridx_v)
    pltpu.sync_copy(tidx_hbm.at[pl.ds(base, bpw)], tidx_v)

    lane = lax.iota(jnp.int32, _LANES)

    def chunk_body(c, carry):
        off = c * _CHUNK
        cp_h = pltpu.async_copy(ent_hbm.at[hidx_v.at[pl.ds(off, _CHUNK)]],
                                hrows, sem)
        cp_r = pltpu.async_copy(rel_hbm.at[ridx_v.at[pl.ds(off, _CHUNK)]],
                                rrows, sem)
        cp_t = pltpu.async_copy(ent_hbm.at[tidx_v.at[pl.ds(off, _CHUNK)]],
                                trows, sem)
        cp_h.wait()
        cp_r.wait()
        cp_t.wait()

        def group_body(g, carry2):
            gbase = g * _LANES
            rows = gbase + lane
            acc = jnp.zeros((_LANES,), jnp.float32)
            for d in range(_D):
                cols = jnp.full((_LANES,), d, jnp.int32)
                h = plsc.load_gather(hrows, [rows, cols])
                r = plsc.load_gather(rrows, [rows, cols])
                t = plsc.load_gather(trows, [rows, cols])
                acc = acc + jnp.abs(h + r - t)
            out_v[pl.ds(off + gbase, _LANES)] = _GAMMA - acc
            return carry2

        lax.fori_loop(0, _CHUNK // _LANES, group_body, 0)
        return carry

    lax.fori_loop(0, bpw // _CHUNK, chunk_body, 0)
    pltpu.sync_copy(out_v, out_hbm.at[pl.ds(base, bpw)])


def kernel(entity_embedding, relation_embedding, sample):
    nc, ns = _sc_geometry()
    nw = nc * ns
    bpw = _B // nw

    heads = sample[:, 0]
    rels = sample[:, 1]
    tails = sample[:, 2]

    mesh = plsc.VectorSubcoreMesh(core_axis_name="c", subcore_axis_name="s")
    kge = functools.partial(
        pl.kernel,
        mesh=mesh,
        compiler_params=pltpu.CompilerParams(needs_layout_passes=False),
        out_type=jax.ShapeDtypeStruct((_B,), jnp.float32),
        scratch_types=[
            pltpu.VMEM((bpw,), jnp.int32),
            pltpu.VMEM((bpw,), jnp.int32),
            pltpu.VMEM((bpw,), jnp.int32),
            pltpu.VMEM((_CHUNK, _D), jnp.float32),
            pltpu.VMEM((_CHUNK, _D), jnp.float32),
            pltpu.VMEM((_CHUNK, _D), jnp.float32),
            pltpu.VMEM((bpw,), jnp.float32),
            pltpu.SemaphoreType.DMA,
        ],
    )(functools.partial(_kge_body, nc, bpw))
    scores = kge(entity_embedding, relation_embedding, heads, rels, tails)
    return scores[:, None]
